# baseline (speedup 1.0000x reference)
import functools

import jax
import jax.numpy as jnp
from jax import lax
from jax.experimental import pallas as pl
from jax.experimental.pallas import tpu as pltpu
from jax.experimental.pallas import tpu_sc as plsc

NH = 4
CODES = 1024
EMB = 1024
HD = EMB // NH
NTOK = 16384
BT = 4096
LOG2E = 1.4426950408889634

_SC_INFO = plsc.get_sparse_core_info()
_NC = _SC_INFO.num_cores
_NS = _SC_INFO.num_subcores
_NW = _NC * _NS
_TPW = NTOK // _NW
_CH = 128
_NCHUNK = _TPW // _CH


def _vq_kernel(scales_ref, z_ref, emb_ref, niota_ref, gidx_ref, idx_ref,
               comb_ref, perp_ref, psum_ref):
    t = pl.program_id(0)
    h = pl.program_id(1)

    @pl.when(jnp.logical_and(t == 0, h == 0))
    def _init():
        psum_ref[...] = jnp.zeros_like(psum_ref)

    z = z_ref[...]
    emb = emb_ref[0]

    zn = z / jnp.maximum(
        jnp.sqrt(jnp.sum(z * z, axis=-1, keepdims=True)), 1e-12)
    en = emb / jnp.maximum(
        jnp.sqrt(jnp.sum(emb * emb, axis=-1, keepdims=True)), 1e-12)

    raw = jax.lax.dot_general(
        zn, en, (((1,), (1,)), ((), ())),
        preferred_element_type=jnp.float32)

    m = jnp.max(raw, axis=-1, keepdims=True)
    niota = niota_ref[...]

    cand = jnp.where(raw >= m, niota, -3.0e38)
    widx = jnp.max(cand, axis=-1, keepdims=True)
    idx = (-widx[:, 0]).astype(jnp.int32)

    k = scales_ref[h] * LOG2E
    e = jnp.exp2(raw * k)
    inv = 1.0 / jnp.sum(e, axis=-1, keepdims=True)
    colsum = jax.lax.dot_general(
        inv, e, (((0,), (0,)), ((), ())),
        preferred_element_type=jnp.float32)
    psum_ref[h, :] = psum_ref[h, :] + colsum[0]

    idx_ref[0, 0, :] = idx
    gidx_ref[0, 0, :] = idx + h * CODES

    @pl.when(h == 0)
    def _comb0():
        comb_ref[0, 0, :] = idx

    @pl.when(h > 0)
    def _combh():
        comb_ref[0, 0, :] = comb_ref[0, 0, :] * CODES + idx

    @pl.when(jnp.logical_and(t == pl.num_programs(0) - 1, h == NH - 1))
    def _finish():
        p = psum_ref[...] * (1.0 / NTOK)
        ent = jnp.sum(p * jnp.log(p + 1e-10), axis=-1, keepdims=True)
        perp_ref[0, 0] = jnp.mean(jnp.exp(-ent))


_SC_MESH = plsc.VectorSubcoreMesh(core_axis_name="c", subcore_axis_name="s")


@functools.partial(
    pl.kernel,
    mesh=_SC_MESH,
    out_type=jax.ShapeDtypeStruct((NTOK, EMB), jnp.float32),
    scratch_types=[
        pltpu.VMEM((_CH,), jnp.int32),
        pltpu.VMEM((_CH, HD), jnp.float32),
        pltpu.SemaphoreType.DMA,
    ],
)
def _sc_gather(table_hbm, gidx_hbm, out_hbm, idx_v, rows_v, sem):
    wid = lax.axis_index("s") * _NC + lax.axis_index("c")
    base = wid * _TPW
    for h in range(NH):
        for c in range(_NCHUNK):
            lo = base + c * _CH
            pltpu.sync_copy(gidx_hbm.at[h, 0, pl.ds(lo, _CH)], idx_v)
            pltpu.async_copy(table_hbm.at[idx_v], rows_v, sem).wait()
            pltpu.sync_copy(rows_v, out_hbm.at[pl.ds(lo, _CH),
                                               pl.ds(h * HD, HD)])


@functools.partial(jax.jit, static_argnames=())
def kernel(z_e, embeddings, logit_scales):
    nt = NTOK // BT
    grid = (nt, NH)
    niota = -jnp.arange(CODES, dtype=jnp.float32).reshape(1, CODES)
    gidx, idx, comb, perp = pl.pallas_call(
        _vq_kernel,
        grid=grid,
        in_specs=[
            pl.BlockSpec(memory_space=pltpu.SMEM),
            pl.BlockSpec((BT, HD), lambda t, h: (t, h)),
            pl.BlockSpec((1, CODES, HD), lambda t, h: (h, 0, 0)),
            pl.BlockSpec((1, CODES), lambda t, h: (0, 0)),
        ],
        out_specs=[
            pl.BlockSpec((1, 1, BT), lambda t, h: (h, 0, t)),
            pl.BlockSpec((1, 1, BT), lambda t, h: (h, 0, t)),
            pl.BlockSpec((1, 1, BT), lambda t, h: (0, 0, t)),
            pl.BlockSpec((1, 1), lambda t, h: (0, 0),
                         memory_space=pltpu.SMEM),
        ],
        out_shape=[
            jax.ShapeDtypeStruct((NH, 1, NTOK), jnp.int32),
            jax.ShapeDtypeStruct((NH, 1, NTOK), jnp.int32),
            jax.ShapeDtypeStruct((1, 1, NTOK), jnp.int32),
            jax.ShapeDtypeStruct((1, 1), jnp.float32),
        ],
        scratch_shapes=[pltpu.VMEM((NH, CODES), jnp.float32)],
    )(logit_scales, z_e, embeddings, niota)

    table = embeddings.reshape(NH * CODES, HD)
    zq = _sc_gather(table, gidx)

    temperature = jnp.asarray(1.0, dtype=jnp.float32)
    commitment_loss = jnp.asarray(0.0, dtype=jnp.float32)
    return (zq, comb[0, 0], perp[0, 0], temperature, commitment_loss)


# speedup vs baseline: 1.0580x; 1.0580x over previous
import functools

import jax
import jax.numpy as jnp
from jax import lax
from jax.experimental import pallas as pl
from jax.experimental.pallas import tpu as pltpu
from jax.experimental.pallas import tpu_sc as plsc

NH = 4
CODES = 1024
EMB = 1024
HD = EMB // NH
NTOK = 16384
BT = 4096
LOG2E = 1.4426950408889634

_SC_INFO = plsc.get_sparse_core_info()
_NC = _SC_INFO.num_cores
_NS = _SC_INFO.num_subcores
_NW = _NC * _NS
_TPW = NTOK // _NW
_CH = 128
_NCHUNK = _TPW // _CH


def _vq_kernel(scales_ref, z_ref, emb_ref, niota_ref, gidx_ref, idx_ref,
               comb_ref, perp_ref, psum_ref):
    t = pl.program_id(0)
    h = pl.program_id(1)

    @pl.when(jnp.logical_and(t == 0, h == 0))
    def _init():
        psum_ref[...] = jnp.zeros_like(psum_ref)

    z = z_ref[...]
    emb = emb_ref[0]

    zn = z / jnp.maximum(
        jnp.sqrt(jnp.sum(z * z, axis=-1, keepdims=True)), 1e-12)
    en = emb / jnp.maximum(
        jnp.sqrt(jnp.sum(emb * emb, axis=-1, keepdims=True)), 1e-12)

    raw = jax.lax.dot_general(
        zn, en, (((1,), (1,)), ((), ())),
        preferred_element_type=jnp.float32)

    m = jnp.max(raw, axis=-1, keepdims=True)
    niota = niota_ref[...]

    cand = jnp.where(raw >= m, niota, -3.0e38)
    widx = jnp.max(cand, axis=-1, keepdims=True)
    idx = (-widx[:, 0]).astype(jnp.int32)

    k = scales_ref[h] * LOG2E
    e = jnp.exp2(raw * k)
    inv = 1.0 / jnp.sum(e, axis=-1, keepdims=True)
    colsum = jax.lax.dot_general(
        inv, e, (((0,), (0,)), ((), ())),
        preferred_element_type=jnp.float32)
    psum_ref[h, :] = psum_ref[h, :] + colsum[0]

    idx_ref[0, 0, :] = idx
    gidx_ref[0, 0, :] = idx + h * CODES

    @pl.when(h == 0)
    def _comb0():
        comb_ref[0, 0, :] = idx

    @pl.when(h > 0)
    def _combh():
        comb_ref[0, 0, :] = comb_ref[0, 0, :] * CODES + idx

    @pl.when(jnp.logical_and(t == pl.num_programs(0) - 1, h == NH - 1))
    def _finish():
        p = psum_ref[...] * (1.0 / NTOK)
        ent = jnp.sum(p * jnp.log(p + 1e-10), axis=-1, keepdims=True)
        perp_ref[0, 0] = jnp.mean(jnp.exp(-ent))


_SC_MESH = plsc.VectorSubcoreMesh(core_axis_name="c", subcore_axis_name="s")


@functools.partial(
    pl.kernel,
    mesh=_SC_MESH,
    out_type=jax.ShapeDtypeStruct((NTOK, EMB), jnp.float32),
    scratch_types=[
        pltpu.VMEM((_CH,), jnp.int32),
        pltpu.VMEM((_CH,), jnp.int32),
        pltpu.VMEM((_CH, HD), jnp.float32),
        pltpu.VMEM((_CH, HD), jnp.float32),
        pltpu.SemaphoreType.DMA,
        pltpu.SemaphoreType.DMA,
    ],
)
def _sc_gather(table_hbm, gidx_hbm, out_hbm, idx_a, idx_b, rows_a, rows_b,
               sem_a, sem_b):
    wid = lax.axis_index("s") * _NC + lax.axis_index("c")
    base = wid * _TPW
    idxs = (idx_a, idx_b)
    bufs = (rows_a, rows_b)
    sems = (sem_a, sem_b)

    def chunk_slices(i):
        h, c = divmod(i, _NCHUNK)
        row = pl.ds(base + c * _CH, _CH)
        col = pl.ds(h * HD, HD)
        return h, row, col

    n = NH * _NCHUNK
    h0, row0, col0 = chunk_slices(0)
    pltpu.sync_copy(gidx_hbm.at[h0, 0, row0], idxs[0])
    copy = pltpu.async_copy(table_hbm.at[idxs[0]], bufs[0], sems[0])
    prev = (copy, row0, col0)
    for i in range(1, n):
        s = i % 2
        h, row, col = chunk_slices(i)
        pltpu.sync_copy(gidx_hbm.at[h, 0, row], idxs[s])
        nxt = pltpu.async_copy(table_hbm.at[idxs[s]], bufs[s], sems[s])
        prev[0].wait()
        pltpu.sync_copy(bufs[1 - s], out_hbm.at[prev[1], prev[2]])
        prev = (nxt, row, col)
    prev[0].wait()
    pltpu.sync_copy(bufs[(n - 1) % 2], out_hbm.at[prev[1], prev[2]])


@functools.partial(jax.jit, static_argnames=())
def kernel(z_e, embeddings, logit_scales):
    nt = NTOK // BT
    grid = (nt, NH)
    niota = -jnp.arange(CODES, dtype=jnp.float32).reshape(1, CODES)
    gidx, idx, comb, perp = pl.pallas_call(
        _vq_kernel,
        grid=grid,
        in_specs=[
            pl.BlockSpec(memory_space=pltpu.SMEM),
            pl.BlockSpec((BT, HD), lambda t, h: (t, h)),
            pl.BlockSpec((1, CODES, HD), lambda t, h: (h, 0, 0)),
            pl.BlockSpec((1, CODES), lambda t, h: (0, 0)),
        ],
        out_specs=[
            pl.BlockSpec((1, 1, BT), lambda t, h: (h, 0, t)),
            pl.BlockSpec((1, 1, BT), lambda t, h: (h, 0, t)),
            pl.BlockSpec((1, 1, BT), lambda t, h: (0, 0, t)),
            pl.BlockSpec((1, 1), lambda t, h: (0, 0),
                         memory_space=pltpu.SMEM),
        ],
        out_shape=[
            jax.ShapeDtypeStruct((NH, 1, NTOK), jnp.int32),
            jax.ShapeDtypeStruct((NH, 1, NTOK), jnp.int32),
            jax.ShapeDtypeStruct((1, 1, NTOK), jnp.int32),
            jax.ShapeDtypeStruct((1, 1), jnp.float32),
        ],
        scratch_shapes=[pltpu.VMEM((NH, CODES), jnp.float32)],
    )(logit_scales, z_e, embeddings, niota)

    table = embeddings.reshape(NH * CODES, HD)
    zq = _sc_gather(table, gidx)

    temperature = jnp.asarray(1.0, dtype=jnp.float32)
    commitment_loss = jnp.asarray(0.0, dtype=jnp.float32)
    return (zq, comb[0, 0], perp[0, 0], temperature, commitment_loss)


# speedup vs baseline: 1.0938x; 1.0338x over previous
import functools

import jax
import jax.numpy as jnp
from jax import lax
from jax.experimental import pallas as pl
from jax.experimental.pallas import tpu as pltpu
from jax.experimental.pallas import tpu_sc as plsc

NH = 4
CODES = 1024
EMB = 1024
HD = EMB // NH
NTOK = 16384
BT = 4096
LOG2E = 1.4426950408889634

_SC_INFO = plsc.get_sparse_core_info()
_NC = _SC_INFO.num_cores
_NS = _SC_INFO.num_subcores
_NW = _NC * _NS
_TPW = NTOK // _NW
_CH = 32
_NCHUNK = _TPW // _CH


def _vq_kernel(scales_ref, z_ref, emb_ref, niota_ref, gidx_ref, idx_ref,
               comb_ref, perp_ref, psum_ref):
    t = pl.program_id(0)
    h = pl.program_id(1)

    @pl.when(jnp.logical_and(t == 0, h == 0))
    def _init():
        psum_ref[...] = jnp.zeros_like(psum_ref)

    z = z_ref[...]
    emb = emb_ref[0]

    zn = z / jnp.maximum(
        jnp.sqrt(jnp.sum(z * z, axis=-1, keepdims=True)), 1e-12)
    en = emb / jnp.maximum(
        jnp.sqrt(jnp.sum(emb * emb, axis=-1, keepdims=True)), 1e-12)

    raw = jax.lax.dot_general(
        zn, en, (((1,), (1,)), ((), ())),
        preferred_element_type=jnp.float32)

    m = jnp.max(raw, axis=-1, keepdims=True)
    niota = niota_ref[...]

    cand = jnp.where(raw >= m, niota, -3.0e38)
    widx = jnp.max(cand, axis=-1, keepdims=True)
    idx = (-widx[:, 0]).astype(jnp.int32)

    k = scales_ref[h] * LOG2E
    e = jnp.exp2(raw * k)
    inv = 1.0 / jnp.sum(e, axis=-1, keepdims=True)
    colsum = jax.lax.dot_general(
        inv, e, (((0,), (0,)), ((), ())),
        preferred_element_type=jnp.float32)
    psum_ref[h, :] = psum_ref[h, :] + colsum[0]

    idx_ref[0, 0, :] = idx
    gidx_ref[0, 0, :] = idx + h * CODES

    @pl.when(h == 0)
    def _comb0():
        comb_ref[0, 0, :] = idx

    @pl.when(h > 0)
    def _combh():
        comb_ref[0, 0, :] = comb_ref[0, 0, :] * CODES + idx

    @pl.when(jnp.logical_and(t == pl.num_programs(0) - 1, h == NH - 1))
    def _finish():
        p = psum_ref[...] * (1.0 / NTOK)
        ent = jnp.sum(p * jnp.log(p + 1e-10), axis=-1, keepdims=True)
        perp_ref[0, 0] = jnp.mean(jnp.exp(-ent))


_SC_MESH = plsc.VectorSubcoreMesh(core_axis_name="c", subcore_axis_name="s")


@functools.partial(
    pl.kernel,
    mesh=_SC_MESH,
    out_type=jax.ShapeDtypeStruct((NTOK, EMB), jnp.float32),
    scratch_types=[
        pltpu.VMEM((NH, _TPW), jnp.int32),
        pltpu.VMEM((_CH, EMB), jnp.float32),
        pltpu.VMEM((_CH, EMB), jnp.float32),
        pltpu.SemaphoreType.DMA,
        pltpu.SemaphoreType.DMA,
    ],
)
def _sc_gather(table_hbm, gidx_hbm, out_hbm, idx_v, rows_a, rows_b,
               sem_a, sem_b):
    wid = lax.axis_index("s") * _NC + lax.axis_index("c")
    base = wid * _TPW
    bufs = (rows_a, rows_b)
    sems = (sem_a, sem_b)
    for h in range(NH):
        pltpu.sync_copy(gidx_hbm.at[h, 0, pl.ds(base, _TPW)],
                        idx_v.at[h])

    def fire(i, s):
        cps = []
        for h in range(NH):
            cps.append(pltpu.async_copy(
                table_hbm.at[idx_v.at[h, pl.ds(i * _CH, _CH)]],
                bufs[s].at[:, pl.ds(h * HD, HD)], sems[s]))
        return cps

    prev = fire(0, 0)
    for i in range(1, _NCHUNK):
        s = i % 2
        nxt = fire(i, s)
        for cp in prev:
            cp.wait()
        pltpu.sync_copy(bufs[1 - s],
                        out_hbm.at[pl.ds(base + (i - 1) * _CH, _CH), :])
        prev = nxt
    for cp in prev:
        cp.wait()
    pltpu.sync_copy(bufs[(_NCHUNK - 1) % 2],
                    out_hbm.at[pl.ds(base + (_NCHUNK - 1) * _CH, _CH), :])


@functools.partial(jax.jit, static_argnames=())
def kernel(z_e, embeddings, logit_scales):
    nt = NTOK // BT
    grid = (nt, NH)
    niota = -jnp.arange(CODES, dtype=jnp.float32).reshape(1, CODES)
    gidx, idx, comb, perp = pl.pallas_call(
        _vq_kernel,
        grid=grid,
        in_specs=[
            pl.BlockSpec(memory_space=pltpu.SMEM),
            pl.BlockSpec((BT, HD), lambda t, h: (t, h)),
            pl.BlockSpec((1, CODES, HD), lambda t, h: (h, 0, 0)),
            pl.BlockSpec((1, CODES), lambda t, h: (0, 0)),
        ],
        out_specs=[
            pl.BlockSpec((1, 1, BT), lambda t, h: (h, 0, t)),
            pl.BlockSpec((1, 1, BT), lambda t, h: (h, 0, t)),
            pl.BlockSpec((1, 1, BT), lambda t, h: (0, 0, t)),
            pl.BlockSpec((1, 1), lambda t, h: (0, 0),
                         memory_space=pltpu.SMEM),
        ],
        out_shape=[
            jax.ShapeDtypeStruct((NH, 1, NTOK), jnp.int32),
            jax.ShapeDtypeStruct((NH, 1, NTOK), jnp.int32),
            jax.ShapeDtypeStruct((1, 1, NTOK), jnp.int32),
            jax.ShapeDtypeStruct((1, 1), jnp.float32),
        ],
        scratch_shapes=[pltpu.VMEM((NH, CODES), jnp.float32)],
    )(logit_scales, z_e, embeddings, niota)

    table = embeddings.reshape(NH * CODES, HD)
    zq = _sc_gather(table, gidx)

    temperature = jnp.asarray(1.0, dtype=jnp.float32)
    commitment_loss = jnp.asarray(0.0, dtype=jnp.float32)
    return (zq, comb[0, 0], perp[0, 0], temperature, commitment_loss)
